# baseline (speedup 1.0000x reference)
import functools
import math

import jax
import jax.numpy as jnp
from jax.experimental import pallas as pl
from jax.experimental.pallas import tpu as pltpu

_QLEN = 2048
_KLEN = 2048
_NUM_BUCKETS = 32
_N_HEADS = 16
_BQ = 512
_NQ = _QLEN // _BQ
_K = 8
_PAD = 4352


def _bias_kernel(delta_ref, tT_ref, out_ref, gb_ref, sem_ref):
    h = pl.program_id(0)
    qb = pl.program_id(1)
    p = h * _NQ + qb

    @pl.when(qb == 0)
    def _build_bank():
        d = jax.lax.broadcasted_iota(jnp.int32, (8, _PAD), 1)
        b = jax.lax.broadcasted_iota(jnp.int32, (8, _PAD), 0)
        rp = d - b - (_QLEN - delta_ref[0])
        n = -rp
        half = _NUM_BUCKETS // 2
        max_exact = half // 2
        ret = jnp.where(n < 0, half, 0).astype(jnp.int32)
        na = jnp.abs(n)
        is_small = na < max_exact
        naf = jnp.maximum(na, 1).astype(jnp.float32)
        t = (jnp.log(naf / max_exact) / math.log(32 / max_exact)
             * (half - max_exact)).astype(jnp.int32)
        val_large = jnp.minimum(max_exact + t, half - 1)
        bucket = ret + jnp.where(is_small, na, val_large)
        vals = jnp.zeros((8, _PAD), jnp.float32)
        for bkt in range(_NUM_BUCKETS):
            vals = jnp.where(bucket == bkt, tT_ref[0, 0, bkt], vals)
        hp = jax.lax.rem(h, 3)
        for a in range(_BQ // 8):
            gb_ref[hp, pl.ds(8 * a, 8), :] = (
                jnp.roll(vals, 8 * a, axis=1) if a else vals)

    def _copy(pi):
        hh = pi // _NQ
        qq = jax.lax.rem(pi, _NQ)
        return pltpu.make_async_copy(
            gb_ref.at[jax.lax.rem(hh, 3), :, pl.ds(_QLEN - qq * _BQ, _KLEN)],
            out_ref.at[0, hh, pl.ds(qq * _BQ, _BQ), :],
            sem_ref.at[jax.lax.rem(pi, _K)],
        )

    _copy(p).start()

    @pl.when(p >= _K)
    def _retire():
        _copy(p - _K).wait()

    last = _N_HEADS * _NQ - 1

    @pl.when(p == last)
    def _drain():
        for j in range(_K - 1, -1, -1):
            _copy(last - j).wait()


@jax.jit
def _run(delta, table_t):
    grid_spec = pltpu.PrefetchScalarGridSpec(
        num_scalar_prefetch=1,
        grid=(_N_HEADS, _NQ),
        in_specs=[pl.BlockSpec((1, 1, _NUM_BUCKETS), lambda h, q, *_: (h, 0, 0))],
        out_specs=pl.BlockSpec(memory_space=pl.ANY),
        scratch_shapes=[
            pltpu.VMEM((3, _BQ, _PAD), jnp.float32),
            pltpu.SemaphoreType.DMA((_K,)),
        ],
    )
    return pl.pallas_call(
        _bias_kernel,
        grid_spec=grid_spec,
        out_shape=jax.ShapeDtypeStruct((1, _N_HEADS, _QLEN, _KLEN), jnp.float32),
    )(delta, table_t)


def kernel(qlen, klen, relative_attention_bias):
    qlen = jnp.asarray(qlen, jnp.int32)
    klen = jnp.asarray(klen, jnp.int32)
    delta = ((klen - _KLEN) - (qlen - _QLEN)).reshape(1)
    table_t = relative_attention_bias.T.reshape(_N_HEADS, 1, _NUM_BUCKETS)
    return _run(delta, table_t)
